# Initial kernel scaffold; baseline (speedup 1.0000x reference)
import jax
import jax.numpy as jnp
from jax.experimental import pallas as pl


def kernel(sca_rep, gro_rep, sca_gate_input, gro_gate_input, sca_g_w1, sca_g_b1, sca_g_gamma, sca_g_beta, sca_g_w2, sca_g_b2, gro_g_w1, gro_g_b1, gro_g_gamma, gro_g_beta, gro_g_w2, gro_g_b2, sca_cluster, gro_cluster, sca_experts_w, sca_experts_b, gro_experts_w, gro_experts_b):
    raise NotImplementedError("write your pallas kernel here")


# speedup vs baseline: 4.2511x; 4.2511x over previous
import jax
import jax.numpy as jnp
from jax.experimental import pallas as pl
from jax.experimental.pallas import tpu as pltpu

B = 4096
EMB = 128
GD = 300
GDP = 384
E = 8
T = 12
AE = E * T


def _fused(xs_ref, xg_ref, rs_ref, rg_ref,
           sw1_ref, sb1_ref, sgm_ref, sbt_ref, sw2_ref, sb2_ref,
           gw1_ref, gb1_ref, ggm_ref, gbt_ref, gw2_ref, gb2_ref,
           sct_ref, gct_ref, sew_ref, seb_ref, gew_ref, geb_ref,
           r_ref, s_ref, out_ref):
    def gate(x, w1, b1, gamma, beta, w2, b2):
        h = jnp.dot(x, w1, preferred_element_type=jnp.float32) + b1
        mu = jnp.mean(h, axis=0, keepdims=True)
        var = jnp.mean((h - mu) ** 2, axis=0, keepdims=True)
        h = (h - mu) * (gamma * jax.lax.rsqrt(var + 1e-5)) + beta
        h = jnp.maximum(h, 0.0)
        return jnp.dot(h, w2, preferred_element_type=jnp.float32) + b2

    def half(x, rep, w1, b1, gamma, beta, w2, b2, clu_t, ew, eb):
        ge = gate(x, w1, b1, gamma, beta, w2, b2)
        logit = jnp.dot(ge, clu_t, preferred_element_type=jnp.float32)
        m = jnp.max(logit, axis=-1, keepdims=True)
        ex = jnp.exp(logit - m)
        q = ex / jnp.sum(ex, axis=-1, keepdims=True)
        z = jnp.dot(rep, ew, preferred_element_type=jnp.float32) + eb
        qe = jnp.dot(q, r_ref[...], preferred_element_type=jnp.float32)
        return jnp.dot(qe * z, s_ref[...], preferred_element_type=jnp.float32)

    sca = half(xs_ref[...], rs_ref[...],
               sw1_ref[...], sb1_ref[...], sgm_ref[...], sbt_ref[...],
               sw2_ref[...], sb2_ref[...], sct_ref[...],
               sew_ref[...], seb_ref[...])
    gro = half(xg_ref[...], rg_ref[...],
               gw1_ref[...], gb1_ref[...], ggm_ref[...], gbt_ref[...],
               gw2_ref[...], gb2_ref[...], gct_ref[...],
               gew_ref[...], geb_ref[...])
    out_ref[...] = 0.5 * (sca + gro)


def kernel(sca_rep, gro_rep, sca_gate_input, gro_gate_input,
           sca_g_w1, sca_g_b1, sca_g_gamma, sca_g_beta, sca_g_w2, sca_g_b2,
           gro_g_w1, gro_g_b1, gro_g_gamma, gro_g_beta, gro_g_w2, gro_g_b2,
           sca_cluster, gro_cluster,
           sca_experts_w, sca_experts_b, gro_experts_w, gro_experts_b):
    p = GDP - GD

    def pad1(v):
        return jnp.pad(v, (0, p)).reshape(1, GDP)

    sw1 = jnp.pad(sca_g_w1, ((0, 0), (0, p)))
    gw1 = jnp.pad(gro_g_w1, ((0, 0), (0, p)))
    sw2 = jnp.pad(sca_g_w2, ((0, p), (0, p)))
    gw2 = jnp.pad(gro_g_w2, ((0, p), (0, p)))
    sct = jnp.pad(sca_cluster, ((0, 0), (0, p))).T
    gct = jnp.pad(gro_cluster, ((0, 0), (0, p))).T
    r_mat = jnp.repeat(jnp.eye(E, dtype=jnp.float32), T, axis=1)
    s_mat = jnp.tile(jnp.eye(T, dtype=jnp.float32), (E, 1))

    out = pl.pallas_call(
        _fused,
        out_shape=jax.ShapeDtypeStruct((B, T), jnp.float32),
        compiler_params=pltpu.CompilerParams(
            vmem_limit_bytes=120 * 1024 * 1024),
    )(sca_gate_input, gro_gate_input, sca_rep, gro_rep,
      sw1, pad1(sca_g_b1), pad1(sca_g_gamma), pad1(sca_g_beta),
      sw2, pad1(sca_g_b2),
      gw1, pad1(gro_g_b1), pad1(gro_g_gamma), pad1(gro_g_beta),
      gw2, pad1(gro_g_b2),
      sct, gct,
      sca_experts_w, sca_experts_b.reshape(1, AE),
      gro_experts_w, gro_experts_b.reshape(1, AE),
      r_mat, s_mat)
    return out


# speedup vs baseline: 6.2502x; 1.4703x over previous
import jax
import jax.numpy as jnp
from jax.experimental import pallas as pl
from jax.experimental.pallas import tpu as pltpu

B = 4096
EMB = 128
GD = 300
E = 8
T = 12
AE = E * T


def _fused(xs_ref, xg_ref, rs_ref, rg_ref,
           sw1_ref, sb1_ref, sgm_ref, sbt_ref, sw2_ref, sb2_ref,
           gw1_ref, gb1_ref, ggm_ref, gbt_ref, gw2_ref, gb2_ref,
           sc_ref, gc_ref, sew_ref, seb_ref, gew_ref, geb_ref,
           r_ref, s_ref, out_ref):
    def gate(x, w1, b1, gamma, beta, w2, b2):
        h = jnp.dot(x, w1, preferred_element_type=jnp.float32) + b1
        mu = jnp.mean(h, axis=0, keepdims=True)
        var = jnp.mean((h - mu) ** 2, axis=0, keepdims=True)
        h = (h - mu) * (gamma * jax.lax.rsqrt(var + 1e-5)) + beta
        h = jnp.maximum(h, 0.0)
        return jnp.dot(h, w2, preferred_element_type=jnp.float32) + b2

    def half(x, rep, w1, b1, gamma, beta, w2, b2, clu, ew, eb):
        ge = gate(x, w1, b1, gamma, beta, w2, b2)
        logit = jax.lax.dot_general(
            ge, clu, (((1,), (1,)), ((), ())),
            preferred_element_type=jnp.float32)
        m = jnp.max(logit, axis=-1, keepdims=True)
        ex = jnp.exp(logit - m)
        q = ex / jnp.sum(ex, axis=-1, keepdims=True)
        z = jnp.dot(rep, ew, preferred_element_type=jnp.float32) + eb
        qe = jnp.dot(q, r_ref[...], preferred_element_type=jnp.float32)
        return jnp.dot(qe * z, s_ref[...], preferred_element_type=jnp.float32)

    sca = half(xs_ref[...], rs_ref[...],
               sw1_ref[...], sb1_ref[...], sgm_ref[...], sbt_ref[...],
               sw2_ref[...], sb2_ref[...], sc_ref[...],
               sew_ref[...], seb_ref[...])
    gro = half(xg_ref[...], rg_ref[...],
               gw1_ref[...], gb1_ref[...], ggm_ref[...], gbt_ref[...],
               gw2_ref[...], gb2_ref[...], gc_ref[...],
               gew_ref[...], geb_ref[...])
    out_ref[...] = 0.5 * (sca + gro)


def kernel(sca_rep, gro_rep, sca_gate_input, gro_gate_input,
           sca_g_w1, sca_g_b1, sca_g_gamma, sca_g_beta, sca_g_w2, sca_g_b2,
           gro_g_w1, gro_g_b1, gro_g_gamma, gro_g_beta, gro_g_w2, gro_g_b2,
           sca_cluster, gro_cluster,
           sca_experts_w, sca_experts_b, gro_experts_w, gro_experts_b):
    r_mat = jnp.repeat(jnp.eye(E, dtype=jnp.float32), T, axis=1)
    s_mat = jnp.tile(jnp.eye(T, dtype=jnp.float32), (E, 1))

    out = pl.pallas_call(
        _fused,
        out_shape=jax.ShapeDtypeStruct((B, T), jnp.float32),
        compiler_params=pltpu.CompilerParams(
            vmem_limit_bytes=120 * 1024 * 1024),
    )(sca_gate_input, gro_gate_input, sca_rep, gro_rep,
      sca_g_w1, sca_g_b1, sca_g_gamma, sca_g_beta, sca_g_w2, sca_g_b2,
      gro_g_w1, gro_g_b1, gro_g_gamma, gro_g_beta, gro_g_w2, gro_g_b2,
      sca_cluster, gro_cluster,
      sca_experts_w, sca_experts_b, gro_experts_w, gro_experts_b,
      r_mat, s_mat)
    return out
